# Initial kernel scaffold; baseline (speedup 1.0000x reference)
import functools

import jax
import jax.numpy as jnp
from jax import lax
from jax.experimental import pallas as pl
from jax.experimental.pallas import tpu as pltpu
from jax.experimental.pallas import tpu_sc as plsc

N = 10000
D = 128
E = 320000

NC = 2
NS = 16
NW = NC * NS
B = 128
R = 80
EP = NW * R * B
NP = 10240
RT = NP // NS

_mesh = plsc.VectorSubcoreMesh(
    core_axis_name="c", subcore_axis_name="s", num_cores=NC, num_subcores=NS)


@functools.partial(
    pl.kernel,
    out_type=(jax.ShapeDtypeStruct((NP, 16), jnp.float32),
              jax.ShapeDtypeStruct((NP, 16), jnp.float32)),
    mesh=_mesh,
    scratch_types=[
        pltpu.VMEM((R, B), jnp.int32),
        pltpu.VMEM((B, 16), jnp.float32),
        pltpu.VMEM_SHARED((NP, 16), jnp.float32),
    ],
)
def _deg_kernel(dstb, zeros16, ones16, out0, out1, dst_v, ones_v, acc):
    cid = lax.axis_index("c")
    sid = lax.axis_index("s")
    wid = sid * NC + cid
    r0 = sid * RT
    pltpu.sync_copy(zeros16.at[pl.ds(r0, RT)], acc.at[pl.ds(r0, RT)])
    pltpu.sync_copy(ones16, ones_v)
    pltpu.sync_copy(dstb.at[pl.ds(wid * R, R)], dst_v)
    plsc.subcore_barrier()

    def body(j, carry):
        pltpu.sync_copy(ones_v, acc.at[dst_v.at[j]], add=True)
        return carry

    lax.fori_loop(0, R, body, 0)
    plsc.subcore_barrier()

    @pl.when(cid == 0)
    def _w0():
        pltpu.sync_copy(acc.at[pl.ds(r0, RT)], out0.at[pl.ds(r0, RT)])

    @pl.when(cid == 1)
    def _w1():
        pltpu.sync_copy(acc.at[pl.ds(r0, RT)], out1.at[pl.ds(r0, RT)])


@functools.partial(
    pl.kernel,
    out_type=(jax.ShapeDtypeStruct((NP, D), jnp.float32),
              jax.ShapeDtypeStruct((NP, D), jnp.float32)),
    mesh=_mesh,
    scratch_types=[
        pltpu.VMEM((R, B), jnp.int32),
        pltpu.VMEM((R, B), jnp.int32),
        pltpu.VMEM((B, D), jnp.float32),
        pltpu.VMEM_SHARED((NP, D), jnp.float32),
        pltpu.SemaphoreType.DMA,
    ],
)
def _edge_kernel(table, srcb, dstb, zerosf, out0, out1,
                 src_v, dst_v, msg, acc, gsem):
    cid = lax.axis_index("c")
    sid = lax.axis_index("s")
    wid = sid * NC + cid
    r0 = sid * RT
    pltpu.sync_copy(zerosf.at[pl.ds(r0, RT)], acc.at[pl.ds(r0, RT)])
    pltpu.sync_copy(srcb.at[pl.ds(wid * R, R)], src_v)
    pltpu.sync_copy(dstb.at[pl.ds(wid * R, R)], dst_v)
    plsc.subcore_barrier()

    def body(j, carry):
        pltpu.async_copy(table.at[src_v.at[j]], msg, gsem).wait()
        pltpu.sync_copy(msg, acc.at[dst_v.at[j]], add=True)
        return carry

    lax.fori_loop(0, R, body, 0)
    plsc.subcore_barrier()

    @pl.when(cid == 0)
    def _w0():
        pltpu.sync_copy(acc.at[pl.ds(r0, RT)], out0.at[pl.ds(r0, RT)])

    @pl.when(cid == 1)
    def _w1():
        pltpu.sync_copy(acc.at[pl.ds(r0, RT)], out1.at[pl.ds(r0, RT)])


BM = 640


def _dinv(d0_ref, d1_ref):
    deg = d0_ref[:, 0:1] + d1_ref[:, 0:1] + 1.0
    return lax.rsqrt(deg)


def _tc1_body(x_ref, w1_ref, d0_ref, d1_ref, t1_ref):
    t1_ref[:, :] = _dinv(d0_ref, d1_ref) * jnp.dot(
        x_ref[:, :], w1_ref[:, :], preferred_element_type=jnp.float32)


def _tc2_body(s0_ref, s1_ref, t1_ref, d0_ref, d1_ref, b1_ref, w2_ref, t2_ref):
    dinv = _dinv(d0_ref, d1_ref)
    h1 = jnp.maximum(
        dinv * (s0_ref[:, :] + s1_ref[:, :] + t1_ref[:, :]) + b1_ref[:, :],
        0.0)
    t2_ref[:, :] = dinv * jnp.dot(
        h1, w2_ref[:, :], preferred_element_type=jnp.float32)


def _tc3_body(s0_ref, s1_ref, t2_ref, d0_ref, d1_ref, b2_ref, wf_ref, bf_ref,
              x_ref, ws_ref, bs_ref, out_ref):
    dinv = _dinv(d0_ref, d1_ref)
    h2 = jnp.maximum(
        dinv * (s0_ref[:, :] + s1_ref[:, :] + t2_ref[:, :]) + b2_ref[:, :],
        0.0)
    out_ref[:, :] = (
        jnp.dot(h2, wf_ref[:, :], preferred_element_type=jnp.float32)
        + bf_ref[:, :]
        + jnp.dot(x_ref[:, :], ws_ref[:, :], preferred_element_type=jnp.float32)
        + bs_ref[:, :])


_feat_spec = pl.BlockSpec((BM, D), lambda i: (i, 0))
_deg_spec = pl.BlockSpec((BM, 16), lambda i: (i, 0))
_w_spec = pl.BlockSpec((D, D), lambda i: (0, 0))
_b_spec = pl.BlockSpec((1, D), lambda i: (0, 0))
_GRID = (NP // BM,)
_OUT_F32 = jax.ShapeDtypeStruct((NP, D), jnp.float32)

_tc1 = pl.pallas_call(
    _tc1_body, grid=_GRID,
    in_specs=[_feat_spec, _w_spec, _deg_spec, _deg_spec],
    out_specs=_feat_spec, out_shape=_OUT_F32)

_tc2 = pl.pallas_call(
    _tc2_body, grid=_GRID,
    in_specs=[_feat_spec, _feat_spec, _feat_spec, _deg_spec, _deg_spec,
              _b_spec, _w_spec],
    out_specs=_feat_spec, out_shape=_OUT_F32)

_tc3 = pl.pallas_call(
    _tc3_body, grid=_GRID,
    in_specs=[_feat_spec, _feat_spec, _feat_spec, _deg_spec, _deg_spec,
              _b_spec, _w_spec, _b_spec, _feat_spec, _w_spec, _b_spec],
    out_specs=_feat_spec, out_shape=_OUT_F32)


def kernel(x, edge_index, W1, b1, W2, b2, W_skip, b_skip, W_final, b_final):
    f32 = jnp.float32
    src = edge_index[0].astype(jnp.int32)
    dst = edge_index[1].astype(jnp.int32)
    pad = EP - E
    src = jnp.concatenate([src, jnp.full((pad,), N, jnp.int32)])
    dst = jnp.concatenate([dst, jnp.full((pad,), N, jnp.int32)])
    srcb = src.reshape(EP // B, B)
    dstb = dst.reshape(EP // B, B)
    xp = jnp.zeros((NP, D), f32).at[:N, :].set(x)
    zeros16 = jnp.zeros((NP, 16), f32)
    ones16 = jnp.ones((B, 16), f32)
    zerosf = jnp.zeros((NP, D), f32)
    b1r = b1.reshape(1, D)
    b2r = b2.reshape(1, D)
    bfr = b_final.reshape(1, D)
    bsr = b_skip.reshape(1, D)

    d0, d1 = _deg_kernel(dstb, zeros16, ones16)
    t1 = _tc1(xp, W1, d0, d1)
    s10, s11 = _edge_kernel(t1, srcb, dstb, zerosf)
    t2 = _tc2(s10, s11, t1, d0, d1, b1r, W2)
    s20, s21 = _edge_kernel(t2, srcb, dstb, zerosf)
    outp = _tc3(s20, s21, t2, d0, d1, b2r, W_final, bfr, xp, W_skip, bsr)
    return outp[:N]


# speedup vs baseline: 7.7930x; 7.7930x over previous
import functools

import jax
import jax.numpy as jnp
from jax import lax
from jax.experimental import pallas as pl
from jax.experimental.pallas import tpu as pltpu
from jax.experimental.pallas import tpu_sc as plsc

N = 10000
D = 128
E = 320000

NC = 2
NS = 16
NW = NC * NS
B = 128
R = 80
EP = NW * R * B
NP = 10240
RT = NP // NS

_mesh = plsc.VectorSubcoreMesh(
    core_axis_name="c", subcore_axis_name="s", num_cores=NC, num_subcores=NS)


@functools.partial(
    pl.kernel,
    out_type=jax.ShapeDtypeStruct((2 * NP, D), jnp.float32),
    mesh=_mesh,
    scratch_types=[
        pltpu.VMEM((R, B), jnp.int32),
        pltpu.VMEM((B, D), jnp.float32),
        pltpu.VMEM_SHARED((NP, D), jnp.float32),
    ],
)
def _deg_kernel(dstb, zerosf, onesf, out, dst_v, ones_v, acc):
    cid = lax.axis_index("c")
    sid = lax.axis_index("s")
    wid = sid * NC + cid
    r0 = sid * RT
    pltpu.sync_copy(zerosf.at[pl.ds(r0, RT)], acc.at[pl.ds(r0, RT)])
    pltpu.sync_copy(onesf, ones_v)
    pltpu.sync_copy(dstb.at[pl.ds(wid * R, R)], dst_v)
    plsc.subcore_barrier()

    def body(j, carry):
        pltpu.sync_copy(ones_v, acc.at[dst_v.at[j]], add=True)
        return carry

    lax.fori_loop(0, R, body, 0)
    plsc.subcore_barrier()
    pltpu.sync_copy(acc.at[pl.ds(r0, RT)], out.at[pl.ds(cid * NP + r0, RT)])


@functools.partial(
    pl.kernel,
    out_type=jax.ShapeDtypeStruct((2 * NP, D), jnp.float32),
    mesh=_mesh,
    scratch_types=[
        pltpu.VMEM((R, B), jnp.int32),
        pltpu.VMEM((R, B), jnp.int32),
        pltpu.VMEM((B, D), jnp.float32),
        pltpu.VMEM_SHARED((NP, D), jnp.float32),
        pltpu.SemaphoreType.DMA,
    ],
)
def _edge_kernel(table, srcb, dstb, zerosf, out,
                 src_v, dst_v, msg, acc, gsem):
    cid = lax.axis_index("c")
    sid = lax.axis_index("s")
    wid = sid * NC + cid
    r0 = sid * RT
    pltpu.sync_copy(zerosf.at[pl.ds(r0, RT)], acc.at[pl.ds(r0, RT)])
    pltpu.sync_copy(srcb.at[pl.ds(wid * R, R)], src_v)
    pltpu.sync_copy(dstb.at[pl.ds(wid * R, R)], dst_v)
    plsc.subcore_barrier()

    def body(j, carry):
        pltpu.async_copy(table.at[src_v.at[j]], msg, gsem).wait()
        pltpu.sync_copy(msg, acc.at[dst_v.at[j]], add=True)
        return carry

    lax.fori_loop(0, R, body, 0)
    plsc.subcore_barrier()
    pltpu.sync_copy(acc.at[pl.ds(r0, RT)], out.at[pl.ds(cid * NP + r0, RT)])


BM = 640


def _dinv(d0_ref, d1_ref):
    deg = d0_ref[:, 0:1] + d1_ref[:, 0:1] + 1.0
    return lax.rsqrt(deg)


def _tc1_body(x_ref, w1_ref, d0_ref, d1_ref, t1_ref):
    t1_ref[:, :] = _dinv(d0_ref, d1_ref) * jnp.dot(
        x_ref[:, :], w1_ref[:, :], preferred_element_type=jnp.float32)


def _tc2_body(s0_ref, s1_ref, t1_ref, d0_ref, d1_ref, b1_ref, w2_ref, t2_ref):
    dinv = _dinv(d0_ref, d1_ref)
    h1 = jnp.maximum(
        dinv * (s0_ref[:, :] + s1_ref[:, :] + t1_ref[:, :]) + b1_ref[:, :],
        0.0)
    t2_ref[:, :] = dinv * jnp.dot(
        h1, w2_ref[:, :], preferred_element_type=jnp.float32)


def _tc3_body(s0_ref, s1_ref, t2_ref, d0_ref, d1_ref, b2_ref, wf_ref, bf_ref,
              x_ref, ws_ref, bs_ref, out_ref):
    dinv = _dinv(d0_ref, d1_ref)
    h2 = jnp.maximum(
        dinv * (s0_ref[:, :] + s1_ref[:, :] + t2_ref[:, :]) + b2_ref[:, :],
        0.0)
    out_ref[:, :] = (
        jnp.dot(h2, wf_ref[:, :], preferred_element_type=jnp.float32)
        + bf_ref[:, :]
        + jnp.dot(x_ref[:, :], ws_ref[:, :], preferred_element_type=jnp.float32)
        + bs_ref[:, :])


_feat_spec = pl.BlockSpec((BM, D), lambda i: (i, 0))
_deg_spec = pl.BlockSpec((BM, D), lambda i: (i, 0))
_w_spec = pl.BlockSpec((D, D), lambda i: (0, 0))
_b_spec = pl.BlockSpec((1, D), lambda i: (0, 0))
_GRID = (NP // BM,)
_OUT_F32 = jax.ShapeDtypeStruct((NP, D), jnp.float32)

_tc1 = pl.pallas_call(
    _tc1_body, grid=_GRID,
    in_specs=[_feat_spec, _w_spec, _deg_spec, _deg_spec],
    out_specs=_feat_spec, out_shape=_OUT_F32)

_tc2 = pl.pallas_call(
    _tc2_body, grid=_GRID,
    in_specs=[_feat_spec, _feat_spec, _feat_spec, _deg_spec, _deg_spec,
              _b_spec, _w_spec],
    out_specs=_feat_spec, out_shape=_OUT_F32)

_tc3 = pl.pallas_call(
    _tc3_body, grid=_GRID,
    in_specs=[_feat_spec, _feat_spec, _feat_spec, _deg_spec, _deg_spec,
              _b_spec, _w_spec, _b_spec, _feat_spec, _w_spec, _b_spec],
    out_specs=_feat_spec, out_shape=_OUT_F32)


def kernel(x, edge_index, W1, b1, W2, b2, W_skip, b_skip, W_final, b_final):
    f32 = jnp.float32
    src = edge_index[0].astype(jnp.int32)
    dst = edge_index[1].astype(jnp.int32)
    pad = EP - E
    src = jnp.concatenate([src, jnp.full((pad,), N, jnp.int32)])
    dst = jnp.concatenate([dst, jnp.full((pad,), N, jnp.int32)])
    srcb = src.reshape(EP // B, B)
    dstb = dst.reshape(EP // B, B)
    xp = jnp.zeros((NP, D), f32).at[:N, :].set(x)
    onesf = jnp.ones((B, D), f32)
    zerosf = jnp.zeros((NP, D), f32)
    b1r = b1.reshape(1, D)
    b2r = b2.reshape(1, D)
    bfr = b_final.reshape(1, D)
    bsr = b_skip.reshape(1, D)

    dd = _deg_kernel(dstb, zerosf, onesf)
    d0, d1 = dd[:NP], dd[NP:]
    t1 = _tc1(xp, W1, d0, d1)
    s1 = _edge_kernel(t1, srcb, dstb, zerosf)
    t2 = _tc2(s1[:NP], s1[NP:], t1, d0, d1, b1r, W2)
    s2 = _edge_kernel(t2, srcb, dstb, zerosf)
    outp = _tc3(s2[:NP], s2[NP:], t2, d0, d1, b2r, W_final, bfr, xp, W_skip, bsr)
    return outp[:N]


# speedup vs baseline: 8.0414x; 1.0319x over previous
import functools

import jax
import jax.numpy as jnp
from jax import lax
from jax.experimental import pallas as pl
from jax.experimental.pallas import tpu as pltpu
from jax.experimental.pallas import tpu_sc as plsc

N = 10000
D = 128
E = 320000

NC = 2
NS = 16
NW = NC * NS
B = 64
R = 160
EP = NW * R * B
NP = 10240
RT = NP // NS

_mesh = plsc.VectorSubcoreMesh(
    core_axis_name="c", subcore_axis_name="s", num_cores=NC, num_subcores=NS)


@functools.partial(
    pl.kernel,
    out_type=jax.ShapeDtypeStruct((2 * NP, D), jnp.float32),
    mesh=_mesh,
    scratch_types=[
        pltpu.VMEM((B,), jnp.int32),
        pltpu.VMEM((B,), jnp.int32),
        pltpu.VMEM((B, D), jnp.float32),
        pltpu.VMEM_SHARED((NP, D), jnp.float32),
        pltpu.SemaphoreType.DMA,
        pltpu.SemaphoreType.DMA,
        pltpu.SemaphoreType.DMA,
        pltpu.SemaphoreType.DMA,
    ],
)
def _deg_kernel(dst, zerosf, onesf, out,
                id0, id1, ones_v, acc, di0, di1, ss0, ss1):
    cid = lax.axis_index("c")
    sid = lax.axis_index("s")
    wid = sid * NC + cid
    r0 = sid * RT
    e0 = wid * R * B
    pltpu.sync_copy(zerosf.at[pl.ds(r0, RT)], acc.at[pl.ds(r0, RT)])
    pltpu.sync_copy(onesf, ones_v)
    plsc.subcore_barrier()

    pltpu.async_copy(dst.at[pl.ds(e0, B)], id0, di0)
    pltpu.async_copy(dst.at[pl.ds(e0 + B, B)], id1, di1)

    def body(i, carry):
        j0 = 2 * i
        j1 = 2 * i + 1
        pltpu.make_async_copy(dst.at[pl.ds(e0, B)], id0, di0).wait()
        pltpu.async_copy(ones_v, acc.at[id0], ss0, add=True)
        pltpu.make_async_copy(dst.at[pl.ds(e0, B)], id1, di1).wait()
        pltpu.async_copy(ones_v, acc.at[id1], ss1, add=True)
        pltpu.make_async_copy(ones_v, acc.at[id0], ss0).wait()

        @pl.when(j0 + 2 < R)
        def _s0():
            pltpu.async_copy(dst.at[pl.ds(e0 + (j0 + 2) * B, B)], id0, di0)

        pltpu.make_async_copy(ones_v, acc.at[id1], ss1).wait()

        @pl.when(j1 + 2 < R)
        def _s1():
            pltpu.async_copy(dst.at[pl.ds(e0 + (j1 + 2) * B, B)], id1, di1)

        return carry

    lax.fori_loop(0, R // 2, body, 0)
    plsc.subcore_barrier()
    pltpu.sync_copy(acc.at[pl.ds(r0, RT)], out.at[pl.ds(cid * NP + r0, RT)])


@functools.partial(
    pl.kernel,
    out_type=jax.ShapeDtypeStruct((2 * NP, D), jnp.float32),
    mesh=_mesh,
    scratch_types=[
        pltpu.VMEM((B,), jnp.int32),
        pltpu.VMEM((B,), jnp.int32),
        pltpu.VMEM((B,), jnp.int32),
        pltpu.VMEM((B,), jnp.int32),
        pltpu.VMEM((B, D), jnp.float32),
        pltpu.VMEM((B, D), jnp.float32),
        pltpu.VMEM_SHARED((NP, D), jnp.float32),
        pltpu.SemaphoreType.DMA,
        pltpu.SemaphoreType.DMA,
        pltpu.SemaphoreType.DMA,
        pltpu.SemaphoreType.DMA,
        pltpu.SemaphoreType.DMA,
        pltpu.SemaphoreType.DMA,
        pltpu.SemaphoreType.DMA,
        pltpu.SemaphoreType.DMA,
    ],
)
def _edge_kernel(table, src, dst, zerosf, out,
                 is0, is1, id0, id1, m0, m1, acc,
                 si0, si1, di0, di1, gs0, gs1, ss0, ss1):
    cid = lax.axis_index("c")
    sid = lax.axis_index("s")
    wid = sid * NC + cid
    r0 = sid * RT
    e0 = wid * R * B
    pltpu.sync_copy(zerosf.at[pl.ds(r0, RT)], acc.at[pl.ds(r0, RT)])
    plsc.subcore_barrier()

    pltpu.async_copy(src.at[pl.ds(e0, B)], is0, si0)
    pltpu.async_copy(dst.at[pl.ds(e0, B)], id0, di0)
    pltpu.make_async_copy(src.at[pl.ds(e0, B)], is0, si0).wait()
    pltpu.async_copy(table.at[is0], m0, gs0)
    pltpu.async_copy(src.at[pl.ds(e0 + B, B)], is1, si1)

    def body(i, carry):
        j0 = 2 * i
        j1 = 2 * i + 1
        pltpu.make_async_copy(table.at[is0], m0, gs0).wait()

        @pl.when(i > 0)
        def _w1():
            pltpu.make_async_copy(m1, acc.at[id1], ss1).wait()

        pltpu.async_copy(dst.at[pl.ds(e0 + j1 * B, B)], id1, di1)
        pltpu.make_async_copy(src.at[pl.ds(e0, B)], is1, si1).wait()
        pltpu.async_copy(table.at[is1], m1, gs1)
        pltpu.make_async_copy(dst.at[pl.ds(e0, B)], id0, di0).wait()
        pltpu.async_copy(m0, acc.at[id0], ss0, add=True)

        @pl.when(j0 + 2 < R)
        def _s0():
            pltpu.async_copy(src.at[pl.ds(e0 + (j0 + 2) * B, B)], is0, si0)

        pltpu.make_async_copy(table.at[is1], m1, gs1).wait()
        pltpu.make_async_copy(m0, acc.at[id0], ss0).wait()

        @pl.when(j0 + 2 < R)
        def _g0():
            pltpu.async_copy(dst.at[pl.ds(e0 + (j0 + 2) * B, B)], id0, di0)
            pltpu.make_async_copy(src.at[pl.ds(e0, B)], is0, si0).wait()
            pltpu.async_copy(table.at[is0], m0, gs0)

        pltpu.make_async_copy(dst.at[pl.ds(e0, B)], id1, di1).wait()
        pltpu.async_copy(m1, acc.at[id1], ss1, add=True)

        @pl.when(j1 + 2 < R)
        def _s1():
            pltpu.async_copy(src.at[pl.ds(e0 + (j1 + 2) * B, B)], is1, si1)

        return carry

    lax.fori_loop(0, R // 2, body, 0)
    pltpu.make_async_copy(m1, acc.at[id1], ss1).wait()
    plsc.subcore_barrier()
    pltpu.sync_copy(acc.at[pl.ds(r0, RT)], out.at[pl.ds(cid * NP + r0, RT)])


BM = 640


def _dinv(d0_ref, d1_ref):
    deg = d0_ref[:, 0:1] + d1_ref[:, 0:1] + 1.0
    return lax.rsqrt(deg)


def _tc1_body(x_ref, w1_ref, d0_ref, d1_ref, t1_ref):
    t1_ref[:, :] = _dinv(d0_ref, d1_ref) * jnp.dot(
        x_ref[:, :], w1_ref[:, :], preferred_element_type=jnp.float32)


def _tc2_body(s0_ref, s1_ref, t1_ref, d0_ref, d1_ref, b1_ref, w2_ref, t2_ref):
    dinv = _dinv(d0_ref, d1_ref)
    h1 = jnp.maximum(
        dinv * (s0_ref[:, :] + s1_ref[:, :] + t1_ref[:, :]) + b1_ref[:, :],
        0.0)
    t2_ref[:, :] = dinv * jnp.dot(
        h1, w2_ref[:, :], preferred_element_type=jnp.float32)


def _tc3_body(s0_ref, s1_ref, t2_ref, d0_ref, d1_ref, b2_ref, wf_ref, bf_ref,
              x_ref, ws_ref, bs_ref, out_ref):
    dinv = _dinv(d0_ref, d1_ref)
    h2 = jnp.maximum(
        dinv * (s0_ref[:, :] + s1_ref[:, :] + t2_ref[:, :]) + b2_ref[:, :],
        0.0)
    out_ref[:, :] = (
        jnp.dot(h2, wf_ref[:, :], preferred_element_type=jnp.float32)
        + bf_ref[:, :]
        + jnp.dot(x_ref[:, :], ws_ref[:, :], preferred_element_type=jnp.float32)
        + bs_ref[:, :])


_feat_spec = pl.BlockSpec((BM, D), lambda i: (i, 0))
_deg_spec = pl.BlockSpec((BM, D), lambda i: (i, 0))
_w_spec = pl.BlockSpec((D, D), lambda i: (0, 0))
_b_spec = pl.BlockSpec((1, D), lambda i: (0, 0))
_GRID = (NP // BM,)
_OUT_F32 = jax.ShapeDtypeStruct((NP, D), jnp.float32)

_tc1 = pl.pallas_call(
    _tc1_body, grid=_GRID,
    in_specs=[_feat_spec, _w_spec, _deg_spec, _deg_spec],
    out_specs=_feat_spec, out_shape=_OUT_F32)

_tc2 = pl.pallas_call(
    _tc2_body, grid=_GRID,
    in_specs=[_feat_spec, _feat_spec, _feat_spec, _deg_spec, _deg_spec,
              _b_spec, _w_spec],
    out_specs=_feat_spec, out_shape=_OUT_F32)

_tc3 = pl.pallas_call(
    _tc3_body, grid=_GRID,
    in_specs=[_feat_spec, _feat_spec, _feat_spec, _deg_spec, _deg_spec,
              _b_spec, _w_spec, _b_spec, _feat_spec, _w_spec, _b_spec],
    out_specs=_feat_spec, out_shape=_OUT_F32)


def kernel(x, edge_index, W1, b1, W2, b2, W_skip, b_skip, W_final, b_final):
    f32 = jnp.float32
    src = edge_index[0].astype(jnp.int32)
    dst = edge_index[1].astype(jnp.int32)
    pad = EP - E
    src = jnp.concatenate([src, jnp.full((pad,), N, jnp.int32)])
    dst = jnp.concatenate([dst, jnp.full((pad,), N, jnp.int32)])
    xp = jnp.zeros((NP, D), f32).at[:N, :].set(x)
    onesf = jnp.ones((B, D), f32)
    zerosf = jnp.zeros((NP, D), f32)
    b1r = b1.reshape(1, D)
    b2r = b2.reshape(1, D)
    bfr = b_final.reshape(1, D)
    bsr = b_skip.reshape(1, D)

    dd = _deg_kernel(dst, zerosf, onesf)
    d0, d1 = dd[:NP], dd[NP:]
    t1 = _tc1(xp, W1, d0, d1)
    s1 = _edge_kernel(t1, src, dst, zerosf)
    t2 = _tc2(s1[:NP], s1[NP:], t1, d0, d1, b1r, W2)
    s2 = _edge_kernel(t2, src, dst, zerosf)
    outp = _tc3(s2[:NP], s2[NP:], t2, d0, d1, b2r, W_final, bfr, xp, W_skip, bsr)
    return outp[:N]


# speedup vs baseline: 8.7592x; 1.0893x over previous
import functools

import jax
import jax.numpy as jnp
from jax import lax
from jax.experimental import pallas as pl
from jax.experimental.pallas import tpu as pltpu
from jax.experimental.pallas import tpu_sc as plsc

N = 10000
D = 128
E = 320000

NC = 2
NS = 16
NW = NC * NS
B = 64
R = 160
EP = NW * R * B
NP = 10240
RT = NP // NS

_mesh = plsc.VectorSubcoreMesh(
    core_axis_name="c", subcore_axis_name="s", num_cores=NC, num_subcores=NS)


@functools.partial(
    pl.kernel,
    out_type=jax.ShapeDtypeStruct((2 * NP, D), jnp.float32),
    mesh=_mesh,
    scratch_types=[
        pltpu.VMEM((B,), jnp.int32),
        pltpu.VMEM((B,), jnp.int32),
        pltpu.VMEM((B, D), jnp.float32),
        pltpu.VMEM_SHARED((NP, D), jnp.float32),
        pltpu.SemaphoreType.DMA,
        pltpu.SemaphoreType.DMA,
        pltpu.SemaphoreType.DMA,
        pltpu.SemaphoreType.DMA,
    ],
)
def _deg_kernel(dst, zerosf, onesf, out,
                id0, id1, ones_v, acc, di0, di1, ss0, ss1):
    cid = lax.axis_index("c")
    sid = lax.axis_index("s")
    wid = sid * NC + cid
    r0 = sid * RT
    e0 = wid * R * B
    pltpu.sync_copy(zerosf.at[pl.ds(r0, RT)], acc.at[pl.ds(r0, RT)])
    pltpu.sync_copy(onesf, ones_v)
    plsc.subcore_barrier()

    pltpu.async_copy(dst.at[pl.ds(e0, B)], id0, di0)
    pltpu.async_copy(dst.at[pl.ds(e0 + B, B)], id1, di1)

    def body(i, carry):
        j0 = 2 * i
        j1 = 2 * i + 1
        pltpu.make_async_copy(dst.at[pl.ds(e0, B)], id0, di0).wait()
        pltpu.async_copy(ones_v, acc.at[id0], ss0, add=True)
        pltpu.make_async_copy(dst.at[pl.ds(e0, B)], id1, di1).wait()
        pltpu.async_copy(ones_v, acc.at[id1], ss1, add=True)
        pltpu.make_async_copy(ones_v, acc.at[id0], ss0).wait()

        @pl.when(j0 + 2 < R)
        def _s0():
            pltpu.async_copy(dst.at[pl.ds(e0 + (j0 + 2) * B, B)], id0, di0)

        pltpu.make_async_copy(ones_v, acc.at[id1], ss1).wait()

        @pl.when(j1 + 2 < R)
        def _s1():
            pltpu.async_copy(dst.at[pl.ds(e0 + (j1 + 2) * B, B)], id1, di1)

        return carry

    lax.fori_loop(0, R // 2, body, 0)
    plsc.subcore_barrier()
    pltpu.sync_copy(acc.at[pl.ds(r0, RT)], out.at[pl.ds(cid * NP + r0, RT)])


@functools.partial(
    pl.kernel,
    out_type=jax.ShapeDtypeStruct((2 * NP, D), jnp.float32),
    mesh=_mesh,
    scratch_types=[
        [pltpu.VMEM((B,), jnp.int32)] * 4,
        [pltpu.VMEM((B,), jnp.int32)] * 4,
        [pltpu.VMEM((B, D), jnp.float32)] * 4,
        pltpu.VMEM_SHARED((NP, D), jnp.float32),
        [pltpu.SemaphoreType.DMA] * 4,
        [pltpu.SemaphoreType.DMA] * 4,
        [pltpu.SemaphoreType.DMA] * 4,
        [pltpu.SemaphoreType.DMA] * 4,
    ],
)
def _edge_kernel(table, src, dst, zerosf, out,
                 isv, idv, m, acc, si, di, gs, ss):
    cid = lax.axis_index("c")
    sid = lax.axis_index("s")
    wid = sid * NC + cid
    r0 = sid * RT
    e0 = wid * R * B
    pltpu.sync_copy(zerosf.at[pl.ds(r0, RT)], acc.at[pl.ds(r0, RT)])
    plsc.subcore_barrier()

    def stage_src(j, k):
        pltpu.async_copy(src.at[pl.ds(e0 + j * B, B)], isv[k], si[k])

    def stage_dst(j, k):
        pltpu.async_copy(dst.at[pl.ds(e0 + j * B, B)], idv[k], di[k])

    for k in range(4):
        stage_src(k, k)
    stage_dst(0, 0)
    stage_dst(1, 1)
    pltpu.make_async_copy(src.at[pl.ds(e0, B)], isv[0], si[0]).wait()
    pltpu.async_copy(table.at[isv[0]], m[0], gs[0])
    pltpu.make_async_copy(src.at[pl.ds(e0, B)], isv[1], si[1]).wait()
    pltpu.async_copy(table.at[isv[1]], m[1], gs[1])

    def block(i, u):
        j = 4 * i + u
        k = u
        k2 = (u + 2) % 4
        pltpu.make_async_copy(table.at[isv[k]], m[k], gs[k]).wait()

        @pl.when(j + 4 < R)
        def _restage_src():
            stage_src(j + 4, k)

        @pl.when(j >= 2)
        def _drain_s():
            pltpu.make_async_copy(m[k2], acc.at[idv[k2]], ss[k2]).wait()

        @pl.when(j + 2 < R)
        def _next_g():
            stage_dst(j + 2, k2)
            pltpu.make_async_copy(src.at[pl.ds(e0, B)], isv[k2], si[k2]).wait()
            pltpu.async_copy(table.at[isv[k2]], m[k2], gs[k2])

        pltpu.make_async_copy(dst.at[pl.ds(e0, B)], idv[k], di[k]).wait()
        pltpu.async_copy(m[k], acc.at[idv[k]], ss[k], add=True)

    def body(i, carry):
        for u in range(4):
            block(i, u)
        return carry

    lax.fori_loop(0, R // 4, body, 0)
    pltpu.make_async_copy(m[(R - 2) % 4], acc.at[idv[(R - 2) % 4]],
                          ss[(R - 2) % 4]).wait()
    pltpu.make_async_copy(m[(R - 1) % 4], acc.at[idv[(R - 1) % 4]],
                          ss[(R - 1) % 4]).wait()
    plsc.subcore_barrier()
    pltpu.sync_copy(acc.at[pl.ds(r0, RT)], out.at[pl.ds(cid * NP + r0, RT)])


BM = 640


def _dinv(d0_ref, d1_ref):
    deg = d0_ref[:, 0:1] + d1_ref[:, 0:1] + 1.0
    return lax.rsqrt(deg)


def _tc1_body(x_ref, w1_ref, d0_ref, d1_ref, t1_ref):
    t1_ref[:, :] = _dinv(d0_ref, d1_ref) * jnp.dot(
        x_ref[:, :], w1_ref[:, :], preferred_element_type=jnp.float32)


def _tc2_body(s0_ref, s1_ref, t1_ref, d0_ref, d1_ref, b1_ref, w2_ref, t2_ref):
    dinv = _dinv(d0_ref, d1_ref)
    h1 = jnp.maximum(
        dinv * (s0_ref[:, :] + s1_ref[:, :] + t1_ref[:, :]) + b1_ref[:, :],
        0.0)
    t2_ref[:, :] = dinv * jnp.dot(
        h1, w2_ref[:, :], preferred_element_type=jnp.float32)


def _tc3_body(s0_ref, s1_ref, t2_ref, d0_ref, d1_ref, b2_ref, wf_ref, bf_ref,
              x_ref, ws_ref, bs_ref, out_ref):
    dinv = _dinv(d0_ref, d1_ref)
    h2 = jnp.maximum(
        dinv * (s0_ref[:, :] + s1_ref[:, :] + t2_ref[:, :]) + b2_ref[:, :],
        0.0)
    out_ref[:, :] = (
        jnp.dot(h2, wf_ref[:, :], preferred_element_type=jnp.float32)
        + bf_ref[:, :]
        + jnp.dot(x_ref[:, :], ws_ref[:, :], preferred_element_type=jnp.float32)
        + bs_ref[:, :])


_feat_spec = pl.BlockSpec((BM, D), lambda i: (i, 0))
_deg_spec = pl.BlockSpec((BM, D), lambda i: (i, 0))
_w_spec = pl.BlockSpec((D, D), lambda i: (0, 0))
_b_spec = pl.BlockSpec((1, D), lambda i: (0, 0))
_GRID = (NP // BM,)
_OUT_F32 = jax.ShapeDtypeStruct((NP, D), jnp.float32)

_tc1 = pl.pallas_call(
    _tc1_body, grid=_GRID,
    in_specs=[_feat_spec, _w_spec, _deg_spec, _deg_spec],
    out_specs=_feat_spec, out_shape=_OUT_F32)

_tc2 = pl.pallas_call(
    _tc2_body, grid=_GRID,
    in_specs=[_feat_spec, _feat_spec, _feat_spec, _deg_spec, _deg_spec,
              _b_spec, _w_spec],
    out_specs=_feat_spec, out_shape=_OUT_F32)

_tc3 = pl.pallas_call(
    _tc3_body, grid=_GRID,
    in_specs=[_feat_spec, _feat_spec, _feat_spec, _deg_spec, _deg_spec,
              _b_spec, _w_spec, _b_spec, _feat_spec, _w_spec, _b_spec],
    out_specs=_feat_spec, out_shape=_OUT_F32)


def kernel(x, edge_index, W1, b1, W2, b2, W_skip, b_skip, W_final, b_final):
    f32 = jnp.float32
    src = edge_index[0].astype(jnp.int32)
    dst = edge_index[1].astype(jnp.int32)
    pad = EP - E
    src = jnp.concatenate([src, jnp.full((pad,), N, jnp.int32)])
    dst = jnp.concatenate([dst, jnp.full((pad,), N, jnp.int32)])
    xp = jnp.zeros((NP, D), f32).at[:N, :].set(x)
    onesf = jnp.ones((B, D), f32)
    zerosf = jnp.zeros((NP, D), f32)
    b1r = b1.reshape(1, D)
    b2r = b2.reshape(1, D)
    bfr = b_final.reshape(1, D)
    bsr = b_skip.reshape(1, D)

    dd = _deg_kernel(dst, zerosf, onesf)
    d0, d1 = dd[:NP], dd[NP:]
    t1 = _tc1(xp, W1, d0, d1)
    s1 = _edge_kernel(t1, src, dst, zerosf)
    t2 = _tc2(s1[:NP], s1[NP:], t1, d0, d1, b1r, W2)
    s2 = _edge_kernel(t2, src, dst, zerosf)
    outp = _tc3(s2[:NP], s2[NP:], t2, d0, d1, b2r, W_final, bfr, xp, W_skip, bsr)
    return outp[:N]


# speedup vs baseline: 9.1609x; 1.0459x over previous
import functools

import jax
import jax.numpy as jnp
from jax import lax
from jax.experimental import pallas as pl
from jax.experimental.pallas import tpu as pltpu
from jax.experimental.pallas import tpu_sc as plsc

N = 10000
D = 128
E = 320000

NC = 2
NS = 16
NW = NC * NS
B = 64
R = 160
R0S = 256
R1S = 64
EP = NW * R * B
NP = 10240
RT = NP // NS

_mesh = plsc.VectorSubcoreMesh(
    core_axis_name="c", subcore_axis_name="s", num_cores=NC, num_subcores=NS)


@functools.partial(
    pl.kernel,
    out_type=jax.ShapeDtypeStruct((2 * NP, D), jnp.float32),
    mesh=_mesh,
    scratch_types=[
        pltpu.VMEM((B,), jnp.int32),
        pltpu.VMEM((B,), jnp.int32),
        pltpu.VMEM((B, D), jnp.float32),
        pltpu.VMEM_SHARED((NP, D), jnp.float32),
        pltpu.SemaphoreType.DMA,
        pltpu.SemaphoreType.DMA,
        pltpu.SemaphoreType.DMA,
        pltpu.SemaphoreType.DMA,
    ],
)
def _deg_kernel(dst, zerosf, onesf, out,
                id0, id1, ones_v, acc, di0, di1, ss0, ss1):
    cid = lax.axis_index("c")
    sid = lax.axis_index("s")
    wid = sid * NC + cid
    r0 = sid * RT
    e0 = wid * R * B
    pltpu.sync_copy(zerosf.at[pl.ds(r0, RT)], acc.at[pl.ds(r0, RT)])
    pltpu.sync_copy(onesf, ones_v)
    plsc.subcore_barrier()

    pltpu.async_copy(dst.at[pl.ds(e0, B)], id0, di0)
    pltpu.async_copy(dst.at[pl.ds(e0 + B, B)], id1, di1)

    def body(i, carry):
        j0 = 2 * i
        j1 = 2 * i + 1
        pltpu.make_async_copy(dst.at[pl.ds(e0, B)], id0, di0).wait()
        pltpu.async_copy(ones_v, acc.at[id0], ss0, add=True)
        pltpu.make_async_copy(dst.at[pl.ds(e0, B)], id1, di1).wait()
        pltpu.async_copy(ones_v, acc.at[id1], ss1, add=True)
        pltpu.make_async_copy(ones_v, acc.at[id0], ss0).wait()

        @pl.when(j0 + 2 < R)
        def _s0():
            pltpu.async_copy(dst.at[pl.ds(e0 + (j0 + 2) * B, B)], id0, di0)

        pltpu.make_async_copy(ones_v, acc.at[id1], ss1).wait()

        @pl.when(j1 + 2 < R)
        def _s1():
            pltpu.async_copy(dst.at[pl.ds(e0 + (j1 + 2) * B, B)], id1, di1)

        return carry

    lax.fori_loop(0, R // 2, body, 0)
    plsc.subcore_barrier()
    pltpu.sync_copy(acc.at[pl.ds(r0, RT)], out.at[pl.ds(cid * NP + r0, RT)])


@functools.partial(
    pl.kernel,
    out_type=jax.ShapeDtypeStruct((2 * NP, D), jnp.float32),
    mesh=_mesh,
    scratch_types=[
        [pltpu.VMEM((B,), jnp.int32)] * 4,
        [pltpu.VMEM((B,), jnp.int32)] * 4,
        [pltpu.VMEM((B, D), jnp.float32)] * 4,
        pltpu.VMEM_SHARED((NP, D), jnp.float32),
        [pltpu.SemaphoreType.DMA] * 4,
        [pltpu.SemaphoreType.DMA] * 4,
        [pltpu.SemaphoreType.DMA] * 4,
        [pltpu.SemaphoreType.DMA] * 4,
    ],
)
def _edge_kernel(table, src, dst, zerosf, out,
                 isv, idv, m, acc, si, di, gs, ss):
    cid = lax.axis_index("c")
    sid = lax.axis_index("s")
    r0 = sid * RT
    rmy = jnp.where(cid == 0, R0S, R1S)
    e0 = (jnp.where(cid == 0, sid * R0S, 16 * R0S + sid * R1S)) * B
    pltpu.sync_copy(zerosf.at[pl.ds(r0, RT)], acc.at[pl.ds(r0, RT)])
    plsc.subcore_barrier()

    def stage_src(j, k):
        pltpu.async_copy(src.at[pl.ds(e0 + j * B, B)], isv[k], si[k])

    def stage_dst(j, k):
        pltpu.async_copy(dst.at[pl.ds(e0 + j * B, B)], idv[k], di[k])

    for k in range(4):
        stage_src(k, k)
    stage_dst(0, 0)
    stage_dst(1, 1)
    pltpu.make_async_copy(src.at[pl.ds(e0, B)], isv[0], si[0]).wait()
    pltpu.async_copy(table.at[isv[0]], m[0], gs[0])
    pltpu.make_async_copy(src.at[pl.ds(e0, B)], isv[1], si[1]).wait()
    pltpu.async_copy(table.at[isv[1]], m[1], gs[1])

    def block(i, u):
        j = 4 * i + u
        k = u
        k2 = (u + 2) % 4
        pltpu.make_async_copy(table.at[isv[k]], m[k], gs[k]).wait()

        @pl.when(j + 4 < rmy)
        def _restage_src():
            stage_src(j + 4, k)

        @pl.when(j >= 2)
        def _drain_s():
            pltpu.make_async_copy(m[k2], acc.at[idv[k2]], ss[k2]).wait()

        @pl.when(j + 2 < rmy)
        def _next_g():
            stage_dst(j + 2, k2)
            pltpu.make_async_copy(src.at[pl.ds(e0, B)], isv[k2], si[k2]).wait()
            pltpu.async_copy(table.at[isv[k2]], m[k2], gs[k2])

        pltpu.make_async_copy(dst.at[pl.ds(e0, B)], idv[k], di[k]).wait()
        pltpu.async_copy(m[k], acc.at[idv[k]], ss[k], add=True)

    def body(i, carry):
        for u in range(4):
            block(i, u)
        return carry

    lax.fori_loop(0, rmy // 4, body, 0)
    pltpu.make_async_copy(m[2], acc.at[idv[2]], ss[2]).wait()
    pltpu.make_async_copy(m[3], acc.at[idv[3]], ss[3]).wait()
    plsc.subcore_barrier()
    pltpu.sync_copy(acc.at[pl.ds(r0, RT)], out.at[pl.ds(cid * NP + r0, RT)])


BM = 640


def _dinv(d0_ref, d1_ref):
    deg = d0_ref[:, 0:1] + d1_ref[:, 0:1] + 1.0
    return lax.rsqrt(deg)


def _tc1_body(x_ref, w1_ref, d0_ref, d1_ref, t1_ref):
    t1_ref[:, :] = _dinv(d0_ref, d1_ref) * jnp.dot(
        x_ref[:, :], w1_ref[:, :], preferred_element_type=jnp.float32)


def _tc2_body(s0_ref, s1_ref, t1_ref, d0_ref, d1_ref, b1_ref, w2_ref, t2_ref):
    dinv = _dinv(d0_ref, d1_ref)
    h1 = jnp.maximum(
        dinv * (s0_ref[:, :] + s1_ref[:, :] + t1_ref[:, :]) + b1_ref[:, :],
        0.0)
    t2_ref[:, :] = dinv * jnp.dot(
        h1, w2_ref[:, :], preferred_element_type=jnp.float32)


def _tc3_body(s0_ref, s1_ref, t2_ref, d0_ref, d1_ref, b2_ref, wf_ref, bf_ref,
              x_ref, ws_ref, bs_ref, out_ref):
    dinv = _dinv(d0_ref, d1_ref)
    h2 = jnp.maximum(
        dinv * (s0_ref[:, :] + s1_ref[:, :] + t2_ref[:, :]) + b2_ref[:, :],
        0.0)
    out_ref[:, :] = (
        jnp.dot(h2, wf_ref[:, :], preferred_element_type=jnp.float32)
        + bf_ref[:, :]
        + jnp.dot(x_ref[:, :], ws_ref[:, :], preferred_element_type=jnp.float32)
        + bs_ref[:, :])


_feat_spec = pl.BlockSpec((BM, D), lambda i: (i, 0))
_deg_spec = pl.BlockSpec((BM, D), lambda i: (i, 0))
_w_spec = pl.BlockSpec((D, D), lambda i: (0, 0))
_b_spec = pl.BlockSpec((1, D), lambda i: (0, 0))
_GRID = (NP // BM,)
_OUT_F32 = jax.ShapeDtypeStruct((NP, D), jnp.float32)

_tc1 = pl.pallas_call(
    _tc1_body, grid=_GRID,
    in_specs=[_feat_spec, _w_spec, _deg_spec, _deg_spec],
    out_specs=_feat_spec, out_shape=_OUT_F32)

_tc2 = pl.pallas_call(
    _tc2_body, grid=_GRID,
    in_specs=[_feat_spec, _feat_spec, _feat_spec, _deg_spec, _deg_spec,
              _b_spec, _w_spec],
    out_specs=_feat_spec, out_shape=_OUT_F32)

_tc3 = pl.pallas_call(
    _tc3_body, grid=_GRID,
    in_specs=[_feat_spec, _feat_spec, _feat_spec, _deg_spec, _deg_spec,
              _b_spec, _w_spec, _b_spec, _feat_spec, _w_spec, _b_spec],
    out_specs=_feat_spec, out_shape=_OUT_F32)


def kernel(x, edge_index, W1, b1, W2, b2, W_skip, b_skip, W_final, b_final):
    f32 = jnp.float32
    src = edge_index[0].astype(jnp.int32)
    dst = edge_index[1].astype(jnp.int32)
    pad = EP - E
    src = jnp.concatenate([src, jnp.full((pad,), N, jnp.int32)])
    dst = jnp.concatenate([dst, jnp.full((pad,), N, jnp.int32)])
    xp = jnp.zeros((NP, D), f32).at[:N, :].set(x)
    onesf = jnp.ones((B, D), f32)
    zerosf = jnp.zeros((NP, D), f32)
    b1r = b1.reshape(1, D)
    b2r = b2.reshape(1, D)
    bfr = b_final.reshape(1, D)
    bsr = b_skip.reshape(1, D)

    dd = _deg_kernel(dst, zerosf, onesf)
    d0, d1 = dd[:NP], dd[NP:]
    t1 = _tc1(xp, W1, d0, d1)
    s1 = _edge_kernel(t1, src, dst, zerosf)
    t2 = _tc2(s1[:NP], s1[NP:], t1, d0, d1, b1r, W2)
    s2 = _edge_kernel(t2, src, dst, zerosf)
    outp = _tc3(s2[:NP], s2[NP:], t2, d0, d1, b2r, W_final, bfr, xp, W_skip, bsr)
    return outp[:N]
